# baseline (speedup 1.0000x reference)
import functools

import jax
import jax.numpy as jnp
from jax import lax
from jax.experimental import pallas as pl
from jax.experimental.pallas import tpu as pltpu
from jax.experimental.pallas import tpu_sc as plsc

D = 32
DP = 128
NC, NS = 2, 16
NW = NC * NS

GW = 128


def _combine_tables(wtT, mT, wuT):
    vocab = wtT.shape[1]
    blk = 1024

    def body(a_ref, m_ref, u_ref, o_ref):
        v = a_ref[...] * m_ref[...] + u_ref[...]
        o_ref[pl.ds(0, blk), pl.ds(0, D)] = jnp.transpose(v, (1, 0))

    in_spec = pl.BlockSpec((D, blk), lambda i: (0, i))
    return pl.pallas_call(
        body,
        grid=(pl.cdiv(vocab, blk),),
        in_specs=[in_spec, in_spec, in_spec],
        out_specs=pl.BlockSpec((blk, DP), lambda i: (i, 0)),
        out_shape=jax.ShapeDtypeStruct((vocab, DP), jnp.float32),
    )(wtT, mT, wuT)


def _make_gather(ntot):
    per_w = ntot // NW
    K = 1280
    steps = per_w // K
    KR = K // GW
    xr_per_w = per_w // GW

    mesh = plsc.VectorSubcoreMesh(core_axis_name="c", subcore_axis_name="s")

    @functools.partial(
        pl.kernel,
        mesh=mesh,
        compiler_params=pltpu.CompilerParams(use_tc_tiling_on_sc=False),
        out_type=jax.ShapeDtypeStruct((ntot, D), jnp.float32),
        scratch_types=[
            pltpu.VMEM((KR, GW), jnp.int32),
            pltpu.VMEM((KR, GW), jnp.int32),
            pltpu.VMEM((K, D), jnp.float32),
            pltpu.VMEM((K, D), jnp.float32),
            pltpu.SemaphoreType.DMA,
            pltpu.SemaphoreType.DMA,
            pltpu.SemaphoreType.DMA,
            pltpu.SemaphoreType.DMA,
        ],
    )
    def gk(x_hbm, v_hbm, out_hbm, i0, i1, b0, b1, g0, g1, w0, w1):
        wid = lax.axis_index("s") * NC + lax.axis_index("c")
        rbase = wid * per_w
        xbase = wid * xr_per_w
        slots = ((i0, b0, g0, w0), (i1, b1, g1, w1))

        @pl.loop(0, steps // 2)
        def _(p):
            for s, (ix, buf, gs, ws) in enumerate(slots):
                g = p * 2 + s

                @pl.when(p > 0)
                def _():
                    pltpu.make_async_copy(
                        buf, out_hbm.at[pl.ds(rbase, K)], ws).wait()

                pltpu.sync_copy(x_hbm.at[pl.ds(xbase + g * KR, KR)], ix)
                for j in range(KR):
                    pltpu.make_async_copy(
                        v_hbm.at[ix.at[j]],
                        buf.at[pl.ds(j * GW, GW)], gs).start()
                for j in range(KR):
                    pltpu.make_async_copy(
                        v_hbm.at[ix.at[j]],
                        buf.at[pl.ds(j * GW, GW)], gs).wait()
                pltpu.make_async_copy(
                    buf, out_hbm.at[pl.ds(rbase + g * K, K)], ws).start()

        for s, (ix, buf, gs, ws) in enumerate(slots):
            pltpu.make_async_copy(
                buf, out_hbm.at[pl.ds(rbase, K)], ws).wait()

    return gk


def _format_out(flat, b, hist):
    x2 = flat.reshape(b, hist * D)

    def body(i_ref, o_ref):
        t = jnp.transpose(i_ref[...], (1, 0))
        o_ref[...] = t.reshape(4, D, b)

    return pl.pallas_call(
        body,
        grid=(hist // 4,),
        in_specs=[pl.BlockSpec((b, 4 * D), lambda i: (0, i))],
        out_specs=pl.BlockSpec((4, D, b), lambda i: (i, 0, 0)),
        out_shape=jax.ShapeDtypeStruct((hist, D, b), jnp.float32),
    )(x2)


def kernel(x, Wt, Wu, M):
    b, h = x.shape
    ntot = b * h
    v = _combine_tables(Wt.T, M.T, Wu.T)
    v4 = v.reshape(4 * v.shape[0], D)
    x4 = x.reshape(ntot // GW, GW) * 4
    out = _make_gather(ntot)(x4, v4)
    out3 = _format_out(out, b, h)
    return jnp.transpose(out3, (2, 0, 1))


# speedup vs baseline: 1.5829x; 1.5829x over previous
import functools

import jax
import jax.numpy as jnp
from jax import lax
from jax.experimental import pallas as pl
from jax.experimental.pallas import tpu as pltpu
from jax.experimental.pallas import tpu_sc as plsc

D = 32
DP = 128
NC, NS = 2, 16
NW = NC * NS

GW = 128


def _combine_tables(wtT, mT, wuT):
    vocab = wtT.shape[1]
    blk = 4096

    def body(a_ref, m_ref, u_ref, o_ref):
        v = a_ref[...] * m_ref[...] + u_ref[...]
        o_ref[pl.ds(0, blk), pl.ds(0, D)] = jnp.transpose(v, (1, 0))

    in_spec = pl.BlockSpec((D, blk), lambda i: (0, i))
    return pl.pallas_call(
        body,
        grid=(pl.cdiv(vocab, blk),),
        in_specs=[in_spec, in_spec, in_spec],
        out_specs=pl.BlockSpec((blk, DP), lambda i: (i, 0)),
        out_shape=jax.ShapeDtypeStruct((vocab, DP), jnp.float32),
    )(wtT, mT, wuT)


def _make_gather(ntot):
    per_w = ntot // NW
    K = 1280
    steps = per_w // K
    KR = K // GW
    xr_per_w = per_w // GW

    mesh = plsc.VectorSubcoreMesh(core_axis_name="c", subcore_axis_name="s")

    @functools.partial(
        pl.kernel,
        mesh=mesh,
        compiler_params=pltpu.CompilerParams(use_tc_tiling_on_sc=False),
        out_type=jax.ShapeDtypeStruct((ntot, D), jnp.float32),
        scratch_types=[
            pltpu.VMEM((KR, GW), jnp.int32),
            pltpu.VMEM((KR, GW), jnp.int32),
            pltpu.VMEM((K, D), jnp.float32),
            pltpu.VMEM((K, D), jnp.float32),
            pltpu.SemaphoreType.DMA,
            pltpu.SemaphoreType.DMA,
            pltpu.SemaphoreType.DMA,
            pltpu.SemaphoreType.DMA,
        ],
    )
    def gk(x_hbm, v_hbm, out_hbm, i0, i1, b0, b1, g0, g1, w0, w1):
        wid = lax.axis_index("s") * NC + lax.axis_index("c")
        rbase = wid * per_w
        xbase = wid * xr_per_w
        slots = ((i0, b0, g0, w0), (i1, b1, g1, w1))

        @pl.loop(0, steps // 2)
        def _(p):
            for s, (ix, buf, gs, ws) in enumerate(slots):
                g = p * 2 + s

                @pl.when(p > 0)
                def _():
                    pltpu.make_async_copy(
                        buf, out_hbm.at[pl.ds(rbase, K)], ws).wait()

                pltpu.sync_copy(x_hbm.at[pl.ds(xbase + g * KR, KR)], ix)
                for j in range(KR):
                    pltpu.make_async_copy(
                        v_hbm.at[ix.at[j]],
                        buf.at[pl.ds(j * GW, GW)], gs).start()
                for j in range(KR):
                    pltpu.make_async_copy(
                        v_hbm.at[ix.at[j]],
                        buf.at[pl.ds(j * GW, GW)], gs).wait()
                pltpu.make_async_copy(
                    buf, out_hbm.at[pl.ds(rbase + g * K, K)], ws).start()

        for s, (ix, buf, gs, ws) in enumerate(slots):
            pltpu.make_async_copy(
                buf, out_hbm.at[pl.ds(rbase, K)], ws).wait()

    return gk


def _format_out(flat, b, hist):
    x2 = flat.reshape(b, hist * D)

    def body(i_ref, o_ref):
        t = jnp.transpose(i_ref[...], (1, 0))
        o_ref[...] = t.reshape(4, D, b)

    return pl.pallas_call(
        body,
        grid=(hist // 4,),
        in_specs=[pl.BlockSpec((b, 4 * D), lambda i: (0, i))],
        out_specs=pl.BlockSpec((4, D, b), lambda i: (i, 0, 0)),
        out_shape=jax.ShapeDtypeStruct((hist, D, b), jnp.float32),
    )(x2)


def kernel(x, Wt, Wu, M):
    b, h = x.shape
    ntot = b * h
    v = _combine_tables(Wt.T, M.T, Wu.T)
    v4 = v.reshape(4 * v.shape[0], D)
    x4 = x.reshape(ntot // GW, GW) * 4
    out = _make_gather(ntot)(x4, v4)
    out3 = _format_out(out, b, h)
    return jnp.transpose(out3, (2, 0, 1))


# speedup vs baseline: 1.7579x; 1.1106x over previous
import functools

import jax
import jax.numpy as jnp
from jax import lax
from jax.experimental import pallas as pl
from jax.experimental.pallas import tpu as pltpu
from jax.experimental.pallas import tpu_sc as plsc

D = 32
DP = 128
NC, NS = 2, 16
NW = NC * NS

GW = 128


def _combine_tables(wtT, mT, wuT):
    vocab = wtT.shape[1]
    blk = 8192

    def body(a_ref, m_ref, u_ref, o_ref):
        v = a_ref[...] * m_ref[...] + u_ref[...]
        o_ref[pl.ds(0, blk), pl.ds(0, D)] = jnp.transpose(v, (1, 0))

    in_spec = pl.BlockSpec((D, blk), lambda i: (0, i))
    return pl.pallas_call(
        body,
        grid=(pl.cdiv(vocab, blk),),
        in_specs=[in_spec, in_spec, in_spec],
        out_specs=pl.BlockSpec((blk, DP), lambda i: (i, 0)),
        out_shape=jax.ShapeDtypeStruct((vocab, DP), jnp.float32),
    )(wtT, mT, wuT)


def _make_gather(ntot):
    per_w = ntot // NW
    K = 1280
    steps = per_w // K
    KR = K // GW
    xr_per_w = per_w // GW

    mesh = plsc.VectorSubcoreMesh(core_axis_name="c", subcore_axis_name="s")

    @functools.partial(
        pl.kernel,
        mesh=mesh,
        compiler_params=pltpu.CompilerParams(use_tc_tiling_on_sc=False),
        out_type=jax.ShapeDtypeStruct((ntot, D), jnp.float32),
        scratch_types=[
            pltpu.VMEM((KR, GW), jnp.int32),
            pltpu.VMEM((KR, GW), jnp.int32),
            pltpu.VMEM((K, D), jnp.float32),
            pltpu.VMEM((K, D), jnp.float32),
            pltpu.SemaphoreType.DMA,
            pltpu.SemaphoreType.DMA,
            pltpu.SemaphoreType.DMA,
            pltpu.SemaphoreType.DMA,
        ],
    )
    def gk(x_hbm, v_hbm, out_hbm, i0, i1, b0, b1, g0, g1, w0, w1):
        wid = lax.axis_index("s") * NC + lax.axis_index("c")
        rbase = wid * per_w
        xbase = wid * xr_per_w
        slots = ((i0, b0, g0, w0), (i1, b1, g1, w1))

        @pl.loop(0, steps // 2)
        def _(p):
            for s, (ix, buf, gs, ws) in enumerate(slots):
                g = p * 2 + s

                @pl.when(p > 0)
                def _():
                    pltpu.make_async_copy(
                        buf, out_hbm.at[pl.ds(rbase, K)], ws).wait()

                pltpu.sync_copy(x_hbm.at[pl.ds(xbase + g * KR, KR)], ix)
                for j in range(KR):
                    pltpu.make_async_copy(
                        v_hbm.at[ix.at[j]],
                        buf.at[pl.ds(j * GW, GW)], gs).start()
                for j in range(KR):
                    pltpu.make_async_copy(
                        v_hbm.at[ix.at[j]],
                        buf.at[pl.ds(j * GW, GW)], gs).wait()
                pltpu.make_async_copy(
                    buf, out_hbm.at[pl.ds(rbase + g * K, K)], ws).start()

        for s, (ix, buf, gs, ws) in enumerate(slots):
            pltpu.make_async_copy(
                buf, out_hbm.at[pl.ds(rbase, K)], ws).wait()

    return gk


def _format_out(flat, b, hist):
    x2 = flat.reshape(b, hist * D)

    def body(i_ref, o_ref):
        t = jnp.transpose(i_ref[...], (1, 0))
        o_ref[...] = t.reshape(4, D, b)

    return pl.pallas_call(
        body,
        grid=(hist // 4,),
        in_specs=[pl.BlockSpec((b, 4 * D), lambda i: (0, i))],
        out_specs=pl.BlockSpec((4, D, b), lambda i: (i, 0, 0)),
        out_shape=jax.ShapeDtypeStruct((hist, D, b), jnp.float32),
    )(x2)


def kernel(x, Wt, Wu, M):
    b, h = x.shape
    ntot = b * h
    v = _combine_tables(Wt.T, M.T, Wu.T)
    v4 = v.reshape(4 * v.shape[0], D)
    x4 = x.reshape(ntot // GW, GW) * 4
    out = _make_gather(ntot)(x4, v4)
    out3 = _format_out(out, b, h)
    return jnp.transpose(out3, (2, 0, 1))


# speedup vs baseline: 1.7931x; 1.0200x over previous
import functools

import jax
import jax.numpy as jnp
from jax import lax
from jax.experimental import pallas as pl
from jax.experimental.pallas import tpu as pltpu
from jax.experimental.pallas import tpu_sc as plsc

D = 32
DP = 128
NC, NS = 2, 16
NW = NC * NS

GW = 128


def _combine_tables(wtT, mT, wuT):
    vocab = wtT.shape[1]
    blk = 16384

    def body(a_ref, m_ref, u_ref, o_ref):
        v = a_ref[...] * m_ref[...] + u_ref[...]
        o_ref[pl.ds(0, blk), pl.ds(0, D)] = jnp.transpose(v, (1, 0))

    in_spec = pl.BlockSpec((D, blk), lambda i: (0, i))
    return pl.pallas_call(
        body,
        grid=(pl.cdiv(vocab, blk),),
        in_specs=[in_spec, in_spec, in_spec],
        out_specs=pl.BlockSpec((blk, DP), lambda i: (i, 0)),
        out_shape=jax.ShapeDtypeStruct((vocab, DP), jnp.float32),
    )(wtT, mT, wuT)


def _make_gather(ntot):
    per_w = ntot // NW
    K = 1280
    steps = per_w // K
    KR = K // GW
    xr_per_w = per_w // GW

    mesh = plsc.VectorSubcoreMesh(core_axis_name="c", subcore_axis_name="s")

    @functools.partial(
        pl.kernel,
        mesh=mesh,
        compiler_params=pltpu.CompilerParams(use_tc_tiling_on_sc=False),
        out_type=jax.ShapeDtypeStruct((ntot, D), jnp.float32),
        scratch_types=[
            pltpu.VMEM((KR, GW), jnp.int32),
            pltpu.VMEM((KR, GW), jnp.int32),
            pltpu.VMEM((K, D), jnp.float32),
            pltpu.VMEM((K, D), jnp.float32),
            pltpu.SemaphoreType.DMA,
            pltpu.SemaphoreType.DMA,
            pltpu.SemaphoreType.DMA,
            pltpu.SemaphoreType.DMA,
        ],
    )
    def gk(x_hbm, v_hbm, out_hbm, i0, i1, b0, b1, g0, g1, w0, w1):
        wid = lax.axis_index("s") * NC + lax.axis_index("c")
        rbase = wid * per_w
        xbase = wid * xr_per_w
        slots = ((i0, b0, g0, w0), (i1, b1, g1, w1))

        @pl.loop(0, steps // 2)
        def _(p):
            for s, (ix, buf, gs, ws) in enumerate(slots):
                g = p * 2 + s

                @pl.when(p > 0)
                def _():
                    pltpu.make_async_copy(
                        buf, out_hbm.at[pl.ds(rbase, K)], ws).wait()

                pltpu.sync_copy(x_hbm.at[pl.ds(xbase + g * KR, KR)], ix)
                for j in range(KR):
                    pltpu.make_async_copy(
                        v_hbm.at[ix.at[j]],
                        buf.at[pl.ds(j * GW, GW)], gs).start()
                for j in range(KR):
                    pltpu.make_async_copy(
                        v_hbm.at[ix.at[j]],
                        buf.at[pl.ds(j * GW, GW)], gs).wait()
                pltpu.make_async_copy(
                    buf, out_hbm.at[pl.ds(rbase + g * K, K)], ws).start()

        for s, (ix, buf, gs, ws) in enumerate(slots):
            pltpu.make_async_copy(
                buf, out_hbm.at[pl.ds(rbase, K)], ws).wait()

    return gk


def _format_out(flat, b, hist):
    x2 = flat.reshape(b, hist * D)

    def body(i_ref, o_ref):
        t = jnp.transpose(i_ref[...], (1, 0))
        o_ref[...] = t.reshape(4, D, b)

    return pl.pallas_call(
        body,
        grid=(hist // 4,),
        in_specs=[pl.BlockSpec((b, 4 * D), lambda i: (0, i))],
        out_specs=pl.BlockSpec((4, D, b), lambda i: (i, 0, 0)),
        out_shape=jax.ShapeDtypeStruct((hist, D, b), jnp.float32),
    )(x2)


def kernel(x, Wt, Wu, M):
    b, h = x.shape
    ntot = b * h
    v = _combine_tables(Wt.T, M.T, Wu.T)
    v4 = v.reshape(4 * v.shape[0], D)
    x4 = x.reshape(ntot // GW, GW) * 4
    out = _make_gather(ntot)(x4, v4)
    out3 = _format_out(out, b, h)
    return jnp.transpose(out3, (2, 0, 1))
